# Initial kernel scaffold; baseline (speedup 1.0000x reference)
import jax
import jax.numpy as jnp
from jax.experimental import pallas as pl


def kernel(x, edge_index, edge_color, W1v, b1v, W2v, b2v, W1c, b1c, W2c, b2c):
    raise NotImplementedError("write your pallas kernel here")


# speedup vs baseline: 1.9993x; 1.9993x over previous
import functools

import jax
import jax.numpy as jnp
from jax import lax
from jax.experimental import pallas as pl
from jax.experimental.pallas import tpu as pltpu
from jax.experimental.pallas import tpu_sc as plsc

NC = 2
NS = 16
LANES = 16
CHUNK = 128


def _mlp_body(x_ref, w1_ref, b1_ref, w2_ref, b2_ref, o_ref):
    h = jnp.maximum(
        jnp.dot(x_ref[...], w1_ref[...], preferred_element_type=jnp.float32)
        + b1_ref[...], 0.0)
    o_ref[...] = (
        jnp.dot(h, w2_ref[...], preferred_element_type=jnp.float32)
        + b2_ref[...])


def _mlp(xx, w1, b1, w2, b2, blk):
    rows, din = xx.shape
    dh = w1.shape[1]
    dout = w2.shape[1]
    b1r = b1.reshape(1, dh)
    b2r = b2.reshape(1, dout)
    return pl.pallas_call(
        _mlp_body,
        grid=(rows // blk,),
        in_specs=[
            pl.BlockSpec((blk, din), lambda i: (i, 0)),
            pl.BlockSpec((din, dh), lambda i: (0, 0)),
            pl.BlockSpec((1, dh), lambda i: (0, 0)),
            pl.BlockSpec((dh, dout), lambda i: (0, 0)),
            pl.BlockSpec((1, dout), lambda i: (0, 0)),
        ],
        out_specs=pl.BlockSpec((blk, dout), lambda i: (i, 0)),
        out_shape=jax.ShapeDtypeStruct((rows, dout), jnp.float32),
    )(xx, w1, b1r, w2, b2r)


def _add_body(a_ref, b_ref, o_ref):
    o_ref[...] = a_ref[...] + b_ref[...]


def _add(a, b, blk):
    rows, d = a.shape
    return pl.pallas_call(
        _add_body,
        grid=(rows // blk,),
        in_specs=[
            pl.BlockSpec((blk, d), lambda i: (i, 0)),
            pl.BlockSpec((blk, d), lambda i: (i, 0)),
        ],
        out_specs=pl.BlockSpec((blk, d), lambda i: (i, 0)),
        out_shape=jax.ShapeDtypeStruct((rows, d), jnp.float32),
    )(a, b)


def _sc_aggregate(hv, src, dst, hc, n_acc):
    n_nodes, d = hv.shape
    e_pad = src.shape[0]
    n_chunks = e_pad // (NC * NS * CHUNK)
    per_worker = n_chunks * CHUNK
    stripe = n_acc // NS

    mesh = plsc.VectorSubcoreMesh(core_axis_name="c", subcore_axis_name="s")

    @functools.partial(
        pl.kernel,
        mesh=mesh,
        out_type=jax.ShapeDtypeStruct((NC, n_acc, d), jnp.float32),
        scratch_types=[
            pltpu.VMEM((CHUNK,), jnp.int32),
            pltpu.VMEM((CHUNK,), jnp.int32),
            pltpu.VMEM((CHUNK, d), jnp.float32),
            pltpu.VMEM((CHUNK, d), jnp.float32),
            pltpu.VMEM_SHARED((n_acc, d), jnp.float32),
            pltpu.SemaphoreType.DMA,
        ],
    )
    def body(hv_hbm, src_hbm, dst_hbm, hc_hbm, out_hbm,
             src_v, dst_v, rows_v, hc_v, acc_sh, sem):
        cid = lax.axis_index("c")
        sid = lax.axis_index("s")
        wid = cid * NS + sid

        def _zrow(r, carry):
            for j in range(d // LANES):
                rows_v[r, pl.ds(j * LANES, LANES)] = jnp.zeros(
                    (LANES,), jnp.float32)
            return carry
        lax.fori_loop(0, CHUNK, _zrow, 0)
        for k in range(stripe // CHUNK):
            pltpu.sync_copy(rows_v,
                            acc_sh.at[pl.ds(sid * stripe + k * CHUNK, CHUNK)])
        plsc.subcore_barrier()

        base0 = wid * per_worker

        def _chunk(ci, carry):
            base = base0 + ci * CHUNK
            pltpu.sync_copy(src_hbm.at[pl.ds(base, CHUNK)], src_v)
            pltpu.sync_copy(dst_hbm.at[pl.ds(base, CHUNK)], dst_v)
            pltpu.async_copy(hv_hbm.at[src_v], rows_v, sem).wait()
            pltpu.sync_copy(hc_hbm.at[pl.ds(base, CHUNK)], hc_v)

            def _addrow(r, c2):
                for j in range(d // LANES):
                    sl = pl.ds(j * LANES, LANES)
                    rows_v[r, sl] = rows_v[r, sl] + hc_v[r, sl]
                return c2
            lax.fori_loop(0, CHUNK, _addrow, 0)
            pltpu.sync_copy(rows_v, acc_sh.at[dst_v], add=True)
            return carry
        lax.fori_loop(0, n_chunks, _chunk, 0)

        plsc.subcore_barrier()

        for k in range(stripe // CHUNK):
            r0 = sid * stripe + k * CHUNK
            pltpu.sync_copy(acc_sh.at[pl.ds(r0, CHUNK)], rows_v)
            pltpu.sync_copy(rows_v, out_hbm.at[cid, pl.ds(r0, CHUNK)])

    return body(hv, src, dst, hc)


def kernel(x, edge_index, edge_color, W1v, b1v, W2v, b2v, W1c, b1c, W2c, b2c):
    n, d = x.shape
    e = edge_index.shape[1]
    dc = edge_color.shape[1]

    src = edge_index[0]
    dst = edge_index[1]

    hv = _mlp(x, W1v, b1v, W2v, b2v, blk=1000)

    epw = NC * NS * CHUNK
    e_pad = ((e + epw - 1) // epw) * epw
    pad = e_pad - e
    ec_pad = jnp.concatenate(
        [edge_color, jnp.zeros((pad, dc), jnp.float32)], axis=0)
    hc = _mlp(ec_pad, W1c, b1c, W2c, b2c, blk=2048)

    src_pad = jnp.concatenate([src, jnp.zeros((pad,), jnp.int32)])
    dst_pad = jnp.concatenate([dst, jnp.full((pad,), n, jnp.int32)])

    n_acc = ((n + NS * CHUNK - 1) // (NS * CHUNK)) * (NS * CHUNK)
    partials = _sc_aggregate(hv, src_pad, dst_pad, hc, n_acc)

    return _add(partials[0, :n], partials[1, :n], blk=1000)
